# Initial kernel scaffold; baseline (speedup 1.0000x reference)
import jax
import jax.numpy as jnp
from jax.experimental import pallas as pl


def kernel(x_num, x_chr, x_strand, x_cas9, x_source, emb_chr, emb_strand, emb_cas9, emb_source, W1, b1, W2, b2):
    raise NotImplementedError("write your pallas kernel here")


# speedup vs baseline: 4.1241x; 4.1241x over previous
import functools

import jax
import jax.numpy as jnp
from jax.experimental import pallas as pl

B = 16384
V_CHR, V_STRAND, V_CAS9, V_SOURCE = 25, 2, 8, 16
BLK = 2048


def _onehot(idx_col, V):
    iota = jax.lax.broadcasted_iota(jnp.int32, (1, V), 1)
    return (idx_col == iota).astype(jnp.float32)


def _mlp_kernel(xnum_ref, ichr_ref, istr_ref, icas_ref, isrc_ref,
                echr_ref, estr_ref, ecas_ref, esrc_ref,
                w1_ref, b1_ref, w2_ref, b2_ref, out_ref):
    w1 = w1_ref[...]
    f32 = jnp.float32
    f_chr = jnp.dot(echr_ref[...], w1[2:10, :], preferred_element_type=f32)
    f_str = jnp.dot(estr_ref[...], w1[10:12, :], preferred_element_type=f32)
    f_cas = jnp.dot(ecas_ref[...], w1[12:20, :], preferred_element_type=f32)
    f_src = jnp.dot(esrc_ref[...], w1[20:28, :], preferred_element_type=f32)

    h = jnp.dot(xnum_ref[...], w1[0:2, :], preferred_element_type=f32)
    h = h + jnp.dot(_onehot(ichr_ref[...], V_CHR), f_chr, preferred_element_type=f32)
    h = h + jnp.dot(_onehot(istr_ref[...], V_STRAND), f_str, preferred_element_type=f32)
    h = h + jnp.dot(_onehot(icas_ref[...], V_CAS9), f_cas, preferred_element_type=f32)
    h = h + jnp.dot(_onehot(isrc_ref[...], V_SOURCE), f_src, preferred_element_type=f32)
    h = jnp.maximum(h + b1_ref[...], 0.0)
    out_ref[...] = jnp.dot(h, w2_ref[...], preferred_element_type=f32) + b2_ref[...]


@jax.jit
def _run(x_num, ichr, istr, icas, isrc,
         emb_chr, emb_strand, emb_cas9, emb_source, W1, b1, W2, b2):
    grid = (B // BLK,)
    data = lambda i: (i, 0)
    full = lambda i: (0, 0)
    return pl.pallas_call(
        _mlp_kernel,
        grid=grid,
        in_specs=[
            pl.BlockSpec((BLK, 2), data),
            pl.BlockSpec((BLK, 1), data),
            pl.BlockSpec((BLK, 1), data),
            pl.BlockSpec((BLK, 1), data),
            pl.BlockSpec((BLK, 1), data),
            pl.BlockSpec((V_CHR, 8), full),
            pl.BlockSpec((V_STRAND, 2), full),
            pl.BlockSpec((V_CAS9, 8), full),
            pl.BlockSpec((V_SOURCE, 8), full),
            pl.BlockSpec((28, 64), full),
            pl.BlockSpec((1, 64), full),
            pl.BlockSpec((64, 32), full),
            pl.BlockSpec((1, 32), full),
        ],
        out_specs=pl.BlockSpec((BLK, 32), data),
        out_shape=jax.ShapeDtypeStruct((B, 32), jnp.float32),
    )(x_num, ichr, istr, icas, isrc,
      emb_chr, emb_strand, emb_cas9, emb_source, W1, b1, W2, b2)


def kernel(x_num, x_chr, x_strand, x_cas9, x_source,
           emb_chr, emb_strand, emb_cas9, emb_source,
           W1, b1, W2, b2):
    ichr = x_chr.astype(jnp.int32).reshape(B, 1)
    istr = x_strand.astype(jnp.int32).reshape(B, 1)
    icas = x_cas9.astype(jnp.int32).reshape(B, 1)
    isrc = x_source.astype(jnp.int32).reshape(B, 1)
    return _run(x_num, ichr, istr, icas, isrc,
                emb_chr, emb_strand, emb_cas9, emb_source,
                W1, b1.reshape(1, 64), W2, b2.reshape(1, 32))
